# baseline (speedup 1.0000x reference)
import functools

import jax
import jax.numpy as jnp
from jax import lax
from jax.experimental import pallas as pl
from jax.experimental.pallas import tpu as pltpu
from jax.experimental.pallas import tpu_sc as plsc

N = 10000
E = 320000
IN = 128
H = 8
F = 32
H2 = H // 2

KB = 64
NBLK = E // KB
NSUB = 16
ROWW = 144
PCH = 50
NCH = N // PCH
NITER = NBLK // NSUB + 2

_R = 400
_G = N // _R


def _tc_body(feat_ref, ah_ref, wfc_ref, q0_ref, q1_ref, w1_ref, w2_ref,
             b1_ref, b2_ref, fs_ref, t_ref):
    x = feat_ref[...]
    fs = lax.dot_general(x, wfc_ref[...], (((1,), (1,)), ((), ())),
                         preferred_element_type=jnp.float32)
    fs_ref[0] = fs[:, :128]
    fs_ref[1] = fs[:, 128:]
    h1 = lax.dot_general(ah_ref[...], w1_ref[...], (((1,), (1,)), ((), ())),
                         preferred_element_type=jnp.float32) + b1_ref[...]
    h1 = jnp.where(h1 > 0, h1, jnp.exp(h1) - 1.0)
    z = lax.dot_general(h1, w2_ref[...], (((1,), (0,)), ((), ())),
                        preferred_element_type=jnp.float32) + b2_ref[...]
    sig = 1.0 / (1.0 + jnp.exp(-z))
    cm8 = (lax.broadcasted_iota(jnp.int32, (1, 16), 1) == 8).astype(jnp.float32)
    ftp8 = sig * cm8
    t_ref[0] = lax.dot_general(fs, q0_ref[...], (((1,), (0,)), ((), ())),
                               preferred_element_type=jnp.float32) + ftp8
    t_ref[1] = lax.dot_general(fs, q1_ref[...], (((1,), (0,)), ((), ())),
                               preferred_element_type=jnp.float32) + ftp8


_tc_call = pl.pallas_call(
    _tc_body,
    grid=(_G,),
    in_specs=[
        pl.BlockSpec((_R, IN), lambda i: (i, 0)),
        pl.BlockSpec((_R, 128), lambda i: (i, 0)),
        pl.BlockSpec((H * F, IN), lambda i: (0, 0)),
        pl.BlockSpec((H * F, 16), lambda i: (0, 0)),
        pl.BlockSpec((H * F, 16), lambda i: (0, 0)),
        pl.BlockSpec((F, 128), lambda i: (0, 0)),
        pl.BlockSpec((F, 16), lambda i: (0, 0)),
        pl.BlockSpec((1, F), lambda i: (0, 0)),
        pl.BlockSpec((1, 16), lambda i: (0, 0)),
    ],
    out_specs=[
        pl.BlockSpec((2, _R, 128), lambda i: (0, i, 0)),
        pl.BlockSpec((2, _R, 16), lambda i: (0, i, 0)),
    ],
    out_shape=[
        jax.ShapeDtypeStruct((2, N, 128), jnp.float32),
        jax.ShapeDtypeStruct((2, N, 16), jnp.float32),
    ],
)


def _sc_body(fstack, tstack, src_hbm, dst_hbm, bias_hbm,
             rst_hbm, alpha_hbm,
             idx_s, idx_d, idx_da, scd, ts, td, fsrc, stage, wblk,
             avblk, biasv, u_acc,
             sem_g0, sem_g1, sem_f0, sem_f1, sem_sc0, sem_sc1,
             sem_al0, sem_al1, sem_ix0, sem_ix1):
    c = lax.axis_index("c")
    s = lax.axis_index("s")
    cN = c * N
    c_f = c.astype(jnp.float32)
    sgn = 1.0 - 2.0 * c_f
    z16 = jnp.zeros((16,), jnp.float32)
    iota16 = lax.broadcasted_iota(jnp.int32, (16,), 0)
    semg = (sem_g0, sem_g1)
    semf = (sem_f0, sem_f1)
    semsc = (sem_sc0, sem_sc1)
    semal = (sem_al0, sem_al1)
    semix = (sem_ix0, sem_ix1)

    def _zrow(i, _):
        for p in range(2):
            for ch in range(ROWW // 16):
                stage[p, i, pl.ds(ch * 16, 16)] = z16
        return 0
    lax.fori_loop(0, KB, _zrow, 0)

    nch = jnp.where(s < NCH - (NCH // NSUB) * NSUB,
                    NCH // NSUB + 1, NCH // NSUB)

    def _zchunk(i, _):
        pltpu.sync_copy(stage.at[0, pl.ds(0, PCH)],
                        u_acc.at[pl.ds((s + i * NSUB) * PCH, PCH)])
        return 0
    lax.fori_loop(0, nch, _zchunk, 0)
    pltpu.sync_copy(bias_hbm.at[pl.ds(c * 128, 128)], biasv)
    plsc.subcore_barrier()

    def blk_of(j):
        return s + j * NSUB

    def idx_prefetch(j, q):
        off = blk_of(j) * KB
        pltpu.async_copy(src_hbm.at[pl.ds(off, KB)], idx_s.at[q], semix[q])
        pltpu.async_copy(dst_hbm.at[pl.ds(off, KB)], idx_d.at[q], semix[q])

    def gather_descs(q):
        return (
            pltpu.make_async_copy(tstack.at[idx_s.at[q]], ts.at[q], semg[q]),
            pltpu.make_async_copy(tstack.at[idx_da.at[q]], td.at[q], semg[q]),
            pltpu.make_async_copy(fstack.at[idx_s.at[q]], fsrc.at[q], semf[q]),
        )

    def fetch(j, q):
        @pl.when((j >= 2) & (blk_of(j - 2) < NBLK))
        def _():
            pltpu.make_async_copy(stage.at[q], u_acc.at[scd.at[q]],
                                  semsc[q]).wait()

            @pl.when(c == 0)
            def _():
                pltpu.make_async_copy(
                    avblk.at[q],
                    alpha_hbm.at[pl.ds(blk_of(j - 2) * KB, KB)],
                    semal[q]).wait()

        @pl.when(blk_of(j) < NBLK)
        def _():
            off = blk_of(j) * KB
            pltpu.make_async_copy(src_hbm.at[pl.ds(off, KB)], idx_s.at[q],
                                  semix[q]).wait()
            pltpu.make_async_copy(dst_hbm.at[pl.ds(off, KB)], idx_d.at[q],
                                  semix[q]).wait()
            for g in range(KB // 16):
                sl = pl.ds(g * 16, 16)
                idx_s[q, sl] = idx_s[q, sl] + cN
                idx_da[q, sl] = idx_d[q, sl] + cN
            for d in gather_descs(q):
                d.start()

        @pl.when(blk_of(j + 1) < NBLK)
        def _():
            idx_prefetch(j + 1, q ^ 1)

    def halfiter(i, p):
        b = blk_of(i)

        @pl.when(b < NBLK)
        def _():
            gd = gather_descs(p)
            gd[0].wait()
            gd[1].wait()
            for g in range(KB // 16):
                lanes = g * 16 + iota16
                fps = plsc.load_gather(ts.at[p], [lanes, jnp.full((16,), 8, jnp.int32)])
                fpd = plsc.load_gather(td.at[p], [lanes, jnp.full((16,), 8, jnp.int32)])
                sig = 1.0 / (1.0 + jnp.exp(-(fps + fpd)))
                av = c_f + sgn * sig
                avblk[p, pl.ds(g * 16, 16)] = sig
                for h in range(H2):
                    el = plsc.load_gather(ts.at[p], [lanes, jnp.full((16,), h, jnp.int32)])
                    er = plsc.load_gather(td.at[p], [lanes, jnp.full((16,), 4 + h, jnp.int32)])
                    x = el + er
                    xlr = jnp.where(x >= 0, x, 0.2 * x)
                    ex = jnp.exp(sgn * xlr)
                    plsc.store_scatter(stage.at[p], [lanes, jnp.full((16,), 128 + h, jnp.int32)], ex)
                    plsc.store_scatter(wblk, [lanes, jnp.full((16,), h, jnp.int32)], ex * av)
                sl = pl.ds(g * 16, 16)
                scd[p, sl] = idx_d[p, sl]

        fetch(i + 1, p ^ 1)

        @pl.when(b < NBLK)
        def _():
            pltpu.make_async_copy(fstack.at[idx_s.at[p]], fsrc.at[p],
                                  semf[p]).wait()
            def _row(r2, _):
                r = r2 * 2
                wva = wblk[r, pl.ds(0, 16)]
                wvb = wblk[r + 1, pl.ds(0, 16)]
                fva = [fsrc[p, r, pl.ds(k * 16, 16)] for k in range(8)]
                fvb = [fsrc[p, r + 1, pl.ds(k * 16, 16)] for k in range(8)]
                for h in range(H2):
                    wa = wva[h]
                    wb = wvb[h]
                    for v in range(2):
                        cl = pl.ds(h * 32 + v * 16, 16)
                        stage[p, r, cl] = fva[h * 2 + v] * wa
                        stage[p, r + 1, cl] = fvb[h * 2 + v] * wb
                return 0
            lax.fori_loop(0, KB // 2, _row, 0)
            pltpu.async_copy(stage.at[p], u_acc.at[scd.at[p]], semsc[p],
                             add=True)

            @pl.when(c == 0)
            def _():
                pltpu.async_copy(avblk.at[p],
                                 alpha_hbm.at[pl.ds(b * KB, KB)], semal[p])

    idx_prefetch(0, 0)
    fetch(0, 0)

    def _pair(t, _):
        halfiter(2 * t, 0)
        halfiter(2 * t + 1, 1)
        return 0
    lax.fori_loop(0, NITER // 2, _pair, 0)
    plsc.subcore_barrier()

    def _fchunk(i, _):
        base = (s + i * NSUB) * PCH
        pltpu.sync_copy(u_acc.at[pl.ds(base, PCH)], stage.at[0, pl.ds(0, PCH)])

        def _node(r, _):
            sv = stage[0, r, pl.ds(128, 16)]
            rv = jnp.where(sv > 0.0, 1.0 / sv, 0.0)
            uv = [stage[0, r, pl.ds(k * 16, 16)] for k in range(8)]
            for h in range(H2):
                rin = rv[h]
                for v in range(2):
                    cl = pl.ds(h * 32 + v * 16, 16)
                    stage[1, r, cl] = uv[h * 2 + v] * rin + biasv[cl]
            return 0
        lax.fori_loop(0, PCH, _node, 0)
        pltpu.sync_copy(stage.at[1, pl.ds(0, PCH), pl.ds(0, 128)],
                        rst_hbm.at[pl.ds(base, PCH), pl.ds(c * 128, 128)])
        return 0
    lax.fori_loop(0, nch, _fchunk, 0)


def _make_sc():
    mesh = plsc.VectorSubcoreMesh(core_axis_name="c", subcore_axis_name="s",
                                  num_cores=2, num_subcores=NSUB)
    return pl.kernel(
        _sc_body,
        out_type=(jax.ShapeDtypeStruct((N, 2 * 128), jnp.float32),
                  jax.ShapeDtypeStruct((E,), jnp.float32)),
        mesh=mesh,
        compiler_params=pltpu.CompilerParams(use_tc_tiling_on_sc=False,
                                             needs_layout_passes=False),
        scratch_types=dict(
            idx_s=pltpu.VMEM((2, KB), jnp.int32),
            idx_d=pltpu.VMEM((2, KB), jnp.int32),
            idx_da=pltpu.VMEM((2, KB), jnp.int32),
            scd=pltpu.VMEM((2, KB), jnp.int32),
            ts=pltpu.VMEM((2, KB, 16), jnp.float32),
            td=pltpu.VMEM((2, KB, 16), jnp.float32),
            fsrc=pltpu.VMEM((2, KB, 128), jnp.float32),
            stage=pltpu.VMEM((2, KB, ROWW), jnp.float32),
            wblk=pltpu.VMEM((KB, 16), jnp.float32),
            avblk=pltpu.VMEM((2, KB), jnp.float32),
            biasv=pltpu.VMEM((128,), jnp.float32),
            u_acc=pltpu.MemorySpace.VMEM_SHARED((N, ROWW), jnp.float32),
            sem_g0=pltpu.SemaphoreType.DMA,
            sem_g1=pltpu.SemaphoreType.DMA,
            sem_f0=pltpu.SemaphoreType.DMA,
            sem_f1=pltpu.SemaphoreType.DMA,
            sem_sc0=pltpu.SemaphoreType.DMA,
            sem_sc1=pltpu.SemaphoreType.DMA,
            sem_al0=pltpu.SemaphoreType.DMA,
            sem_al1=pltpu.SemaphoreType.DMA,
            sem_ix0=pltpu.SemaphoreType.DMA,
            sem_ix1=pltpu.SemaphoreType.DMA,
        ),
    )


def kernel(feat, alpha_hidden, edge_index, W_fc, attn_l, attn_r, attn_ln,
           attn_rn, bias, W1, b1, W2, b2):
    eye4 = jnp.eye(4, dtype=jnp.float32)
    def bd(a):
        return (a[:, :, None] * eye4[:, None, :]).reshape(128, 4)
    q0 = jnp.zeros((H * F, 16), jnp.float32)
    q0 = q0.at[0:128, 0:4].set(bd(attn_l[0])).at[0:128, 4:8].set(bd(attn_r[0]))
    q1 = jnp.zeros((H * F, 16), jnp.float32)
    q1 = q1.at[128:256, 0:4].set(bd(attn_ln[0])).at[128:256, 4:8].set(bd(attn_rn[0]))

    w2p = jnp.zeros((F, 16), jnp.float32).at[:, 8].set(W2[0])
    b2r = jnp.broadcast_to(b2.reshape(1, 1), (1, 16))
    fs2, t2 = _tc_call(feat, alpha_hidden, W_fc, q0, q1, W1,
                       w2p, b1.reshape(1, F), b2r)
    fstack = fs2.reshape(2 * N, 128)
    tstack = t2.reshape(2 * N, 16)

    src = edge_index[0]
    dst = edge_index[1]
    rst_flat, alpha = _make_sc()(fstack, tstack, src, dst, bias)
    return (rst_flat.reshape(N, H, F), alpha.reshape(E, 1, 1))


# speedup vs baseline: 1.0203x; 1.0203x over previous
import functools

import jax
import jax.numpy as jnp
from jax import lax
from jax.experimental import pallas as pl
from jax.experimental.pallas import tpu as pltpu
from jax.experimental.pallas import tpu_sc as plsc

N = 10000
E = 320000
IN = 128
H = 8
F = 32
H2 = H // 2

KB = 64
NBLK = E // KB
NSUB = 16
ROWW = 136
PCH = 50
NCH = N // PCH
NITER = NBLK // NSUB + 2

_R = 400
_G = N // _R


def _tc_body(feat_ref, ah_ref, wfc_ref, q0_ref, q1_ref, w1_ref, w2_ref,
             b1_ref, b2_ref, fs_ref, t_ref):
    x = feat_ref[...]
    fs = lax.dot_general(x, wfc_ref[...], (((1,), (1,)), ((), ())),
                         preferred_element_type=jnp.float32)
    fs_ref[0] = fs[:, :128]
    fs_ref[1] = fs[:, 128:]
    h1 = lax.dot_general(ah_ref[...], w1_ref[...], (((1,), (1,)), ((), ())),
                         preferred_element_type=jnp.float32) + b1_ref[...]
    h1 = jnp.where(h1 > 0, h1, jnp.exp(h1) - 1.0)
    z = lax.dot_general(h1, w2_ref[...], (((1,), (0,)), ((), ())),
                        preferred_element_type=jnp.float32) + b2_ref[...]
    sig = 1.0 / (1.0 + jnp.exp(-z))
    cm8 = (lax.broadcasted_iota(jnp.int32, (1, 16), 1) == 8).astype(jnp.float32)
    ftp8 = sig * cm8
    t_ref[0] = lax.dot_general(fs, q0_ref[...], (((1,), (0,)), ((), ())),
                               preferred_element_type=jnp.float32) + ftp8
    t_ref[1] = lax.dot_general(fs, q1_ref[...], (((1,), (0,)), ((), ())),
                               preferred_element_type=jnp.float32) + ftp8


_tc_call = pl.pallas_call(
    _tc_body,
    grid=(_G,),
    in_specs=[
        pl.BlockSpec((_R, IN), lambda i: (i, 0)),
        pl.BlockSpec((_R, 128), lambda i: (i, 0)),
        pl.BlockSpec((H * F, IN), lambda i: (0, 0)),
        pl.BlockSpec((H * F, 16), lambda i: (0, 0)),
        pl.BlockSpec((H * F, 16), lambda i: (0, 0)),
        pl.BlockSpec((F, 128), lambda i: (0, 0)),
        pl.BlockSpec((F, 16), lambda i: (0, 0)),
        pl.BlockSpec((1, F), lambda i: (0, 0)),
        pl.BlockSpec((1, 16), lambda i: (0, 0)),
    ],
    out_specs=[
        pl.BlockSpec((2, _R, 128), lambda i: (0, i, 0)),
        pl.BlockSpec((2, _R, 16), lambda i: (0, i, 0)),
    ],
    out_shape=[
        jax.ShapeDtypeStruct((2, N, 128), jnp.float32),
        jax.ShapeDtypeStruct((2, N, 16), jnp.float32),
    ],
)


def _sc_body(fstack, tstack, src_hbm, dst_hbm, bias_hbm,
             rst_hbm, alpha_hbm,
             idx_s, idx_d, idx_da, scd, ts, td, fsrc, stage, wblk,
             avblk, biasv, u_acc,
             sem_g0, sem_g1, sem_f0, sem_f1, sem_sc0, sem_sc1,
             sem_al0, sem_al1, sem_ix0, sem_ix1):
    c = lax.axis_index("c")
    s = lax.axis_index("s")
    cN = c * N
    c_f = c.astype(jnp.float32)
    sgn = 1.0 - 2.0 * c_f
    z16 = jnp.zeros((16,), jnp.float32)
    iota16 = lax.broadcasted_iota(jnp.int32, (16,), 0)
    semg = (sem_g0, sem_g1)
    semf = (sem_f0, sem_f1)
    semsc = (sem_sc0, sem_sc1)
    semal = (sem_al0, sem_al1)
    semix = (sem_ix0, sem_ix1)

    def _zrow(i, _):
        for p in range(2):
            for ch in range(8):
                stage[p, i, pl.ds(ch * 16, 16)] = z16
            stage[p, i, pl.ds(ROWW - 16, 16)] = z16
        return 0
    lax.fori_loop(0, KB, _zrow, 0)

    nch = jnp.where(s < NCH - (NCH // NSUB) * NSUB,
                    NCH // NSUB + 1, NCH // NSUB)

    def _zchunk(i, _):
        pltpu.sync_copy(stage.at[0, pl.ds(0, PCH)],
                        u_acc.at[pl.ds((s + i * NSUB) * PCH, PCH)])
        return 0
    lax.fori_loop(0, nch, _zchunk, 0)
    pltpu.sync_copy(bias_hbm.at[pl.ds(c * 128, 128)], biasv)
    plsc.subcore_barrier()

    def blk_of(j):
        return s + j * NSUB

    def idx_prefetch(j, q):
        off = blk_of(j) * KB
        pltpu.async_copy(src_hbm.at[pl.ds(off, KB)], idx_s.at[q], semix[q])
        pltpu.async_copy(dst_hbm.at[pl.ds(off, KB)], idx_d.at[q], semix[q])

    def gather_descs(q):
        return (
            pltpu.make_async_copy(tstack.at[idx_s.at[q]], ts.at[q], semg[q]),
            pltpu.make_async_copy(tstack.at[idx_da.at[q]], td.at[q], semg[q]),
            pltpu.make_async_copy(fstack.at[idx_s.at[q]], fsrc.at[q], semf[q]),
        )

    def fetch(j, q):
        @pl.when((j >= 2) & (blk_of(j - 2) < NBLK))
        def _():
            pltpu.make_async_copy(stage.at[q], u_acc.at[scd.at[q]],
                                  semsc[q]).wait()

            @pl.when(c == 0)
            def _():
                pltpu.make_async_copy(
                    avblk.at[q],
                    alpha_hbm.at[pl.ds(blk_of(j - 2) * KB, KB)],
                    semal[q]).wait()

        @pl.when(blk_of(j) < NBLK)
        def _():
            off = blk_of(j) * KB
            pltpu.make_async_copy(src_hbm.at[pl.ds(off, KB)], idx_s.at[q],
                                  semix[q]).wait()
            pltpu.make_async_copy(dst_hbm.at[pl.ds(off, KB)], idx_d.at[q],
                                  semix[q]).wait()
            for g in range(KB // 16):
                sl = pl.ds(g * 16, 16)
                idx_s[q, sl] = idx_s[q, sl] + cN
                idx_da[q, sl] = idx_d[q, sl] + cN
            for d in gather_descs(q):
                d.start()

        @pl.when(blk_of(j + 1) < NBLK)
        def _():
            idx_prefetch(j + 1, q ^ 1)

    def halfiter(i, p):
        b = blk_of(i)

        @pl.when(b < NBLK)
        def _():
            gd = gather_descs(p)
            gd[0].wait()
            gd[1].wait()
            for g in range(KB // 16):
                lanes = g * 16 + iota16
                fps = plsc.load_gather(ts.at[p], [lanes, jnp.full((16,), 8, jnp.int32)])
                fpd = plsc.load_gather(td.at[p], [lanes, jnp.full((16,), 8, jnp.int32)])
                sig = 1.0 / (1.0 + jnp.exp(-(fps + fpd)))
                av = c_f + sgn * sig
                avblk[p, pl.ds(g * 16, 16)] = sig
                for h in range(H2):
                    el = plsc.load_gather(ts.at[p], [lanes, jnp.full((16,), h, jnp.int32)])
                    er = plsc.load_gather(td.at[p], [lanes, jnp.full((16,), 4 + h, jnp.int32)])
                    x = el + er
                    xlr = jnp.where(x >= 0, x, 0.2 * x)
                    ex = jnp.exp(sgn * xlr)
                    plsc.store_scatter(stage.at[p], [lanes, jnp.full((16,), 128 + h, jnp.int32)], ex)
                    plsc.store_scatter(wblk, [lanes, jnp.full((16,), h, jnp.int32)], ex * av)
                sl = pl.ds(g * 16, 16)
                scd[p, sl] = idx_d[p, sl]

        fetch(i + 1, p ^ 1)

        @pl.when(b < NBLK)
        def _():
            pltpu.make_async_copy(fstack.at[idx_s.at[p]], fsrc.at[p],
                                  semf[p]).wait()
            def _row(r2, _):
                r = r2 * 2
                wva = wblk[r, pl.ds(0, 16)]
                wvb = wblk[r + 1, pl.ds(0, 16)]
                fva = [fsrc[p, r, pl.ds(k * 16, 16)] for k in range(8)]
                fvb = [fsrc[p, r + 1, pl.ds(k * 16, 16)] for k in range(8)]
                for h in range(H2):
                    wa = wva[h]
                    wb = wvb[h]
                    for v in range(2):
                        cl = pl.ds(h * 32 + v * 16, 16)
                        stage[p, r, cl] = fva[h * 2 + v] * wa
                        stage[p, r + 1, cl] = fvb[h * 2 + v] * wb
                return 0
            lax.fori_loop(0, KB // 2, _row, 0)
            pltpu.async_copy(stage.at[p], u_acc.at[scd.at[p]], semsc[p],
                             add=True)

            @pl.when(c == 0)
            def _():
                pltpu.async_copy(avblk.at[p],
                                 alpha_hbm.at[pl.ds(b * KB, KB)], semal[p])

    idx_prefetch(0, 0)
    fetch(0, 0)

    def _pair(t, _):
        halfiter(2 * t, 0)
        halfiter(2 * t + 1, 1)
        return 0
    lax.fori_loop(0, NITER // 2, _pair, 0)
    plsc.subcore_barrier()

    def _fchunk(i, _):
        base = (s + i * NSUB) * PCH
        pltpu.sync_copy(u_acc.at[pl.ds(base, PCH)], stage.at[0, pl.ds(0, PCH)])

        def _node(r, _):
            sv = stage[0, r, pl.ds(ROWW - 16, 16)]
            rv = jnp.where(sv > 0.0, 1.0 / sv, 0.0)
            uv = [stage[0, r, pl.ds(k * 16, 16)] for k in range(8)]
            for h in range(H2):
                rin = rv[8 + h]
                for v in range(2):
                    cl = pl.ds(h * 32 + v * 16, 16)
                    stage[1, r, cl] = uv[h * 2 + v] * rin + biasv[cl]
            return 0
        lax.fori_loop(0, PCH, _node, 0)
        pltpu.sync_copy(stage.at[1, pl.ds(0, PCH), pl.ds(0, 128)],
                        rst_hbm.at[pl.ds(base, PCH), pl.ds(c * 128, 128)])
        return 0
    lax.fori_loop(0, nch, _fchunk, 0)


def _make_sc():
    mesh = plsc.VectorSubcoreMesh(core_axis_name="c", subcore_axis_name="s",
                                  num_cores=2, num_subcores=NSUB)
    return pl.kernel(
        _sc_body,
        out_type=(jax.ShapeDtypeStruct((N, 2 * 128), jnp.float32),
                  jax.ShapeDtypeStruct((E,), jnp.float32)),
        mesh=mesh,
        compiler_params=pltpu.CompilerParams(use_tc_tiling_on_sc=False,
                                             needs_layout_passes=False),
        scratch_types=dict(
            idx_s=pltpu.VMEM((2, KB), jnp.int32),
            idx_d=pltpu.VMEM((2, KB), jnp.int32),
            idx_da=pltpu.VMEM((2, KB), jnp.int32),
            scd=pltpu.VMEM((2, KB), jnp.int32),
            ts=pltpu.VMEM((2, KB, 16), jnp.float32),
            td=pltpu.VMEM((2, KB, 16), jnp.float32),
            fsrc=pltpu.VMEM((2, KB, 128), jnp.float32),
            stage=pltpu.VMEM((2, KB, ROWW), jnp.float32),
            wblk=pltpu.VMEM((KB, 16), jnp.float32),
            avblk=pltpu.VMEM((2, KB), jnp.float32),
            biasv=pltpu.VMEM((128,), jnp.float32),
            u_acc=pltpu.MemorySpace.VMEM_SHARED((N, ROWW), jnp.float32),
            sem_g0=pltpu.SemaphoreType.DMA,
            sem_g1=pltpu.SemaphoreType.DMA,
            sem_f0=pltpu.SemaphoreType.DMA,
            sem_f1=pltpu.SemaphoreType.DMA,
            sem_sc0=pltpu.SemaphoreType.DMA,
            sem_sc1=pltpu.SemaphoreType.DMA,
            sem_al0=pltpu.SemaphoreType.DMA,
            sem_al1=pltpu.SemaphoreType.DMA,
            sem_ix0=pltpu.SemaphoreType.DMA,
            sem_ix1=pltpu.SemaphoreType.DMA,
        ),
    )


def kernel(feat, alpha_hidden, edge_index, W_fc, attn_l, attn_r, attn_ln,
           attn_rn, bias, W1, b1, W2, b2):
    eye4 = jnp.eye(4, dtype=jnp.float32)
    def bd(a):
        return (a[:, :, None] * eye4[:, None, :]).reshape(128, 4)
    q0 = jnp.zeros((H * F, 16), jnp.float32)
    q0 = q0.at[0:128, 0:4].set(bd(attn_l[0])).at[0:128, 4:8].set(bd(attn_r[0]))
    q1 = jnp.zeros((H * F, 16), jnp.float32)
    q1 = q1.at[128:256, 0:4].set(bd(attn_ln[0])).at[128:256, 4:8].set(bd(attn_rn[0]))

    w2p = jnp.zeros((F, 16), jnp.float32).at[:, 8].set(W2[0])
    b2r = jnp.broadcast_to(b2.reshape(1, 1), (1, 16))
    fs2, t2 = _tc_call(feat, alpha_hidden, W_fc, q0, q1, W1,
                       w2p, b1.reshape(1, F), b2r)
    fstack = fs2.reshape(2 * N, 128)
    tstack = t2.reshape(2 * N, 16)

    src = edge_index[0]
    dst = edge_index[1]
    rst_flat, alpha = _make_sc()(fstack, tstack, src, dst, bias)
    return (rst_flat.reshape(N, H, F), alpha.reshape(E, 1, 1))


# speedup vs baseline: 1.0314x; 1.0109x over previous
import functools

import jax
import jax.numpy as jnp
import numpy as np
from jax import lax
from jax.experimental import pallas as pl
from jax.experimental.pallas import tpu as pltpu
from jax.experimental.pallas import tpu_sc as plsc

N = 10000
E = 320000
IN = 128
H = 8
F = 32
H2 = H // 2

KB = 64
NBLK = E // KB
NSUB = 16
ROWW = 136
PCH = 50
NCH = N // PCH
NITER = NBLK // NSUB + 2

_R = 400
_G = N // _R


def _tc_body(feat_ref, ah_ref, wfc_ref, q0_ref, q1_ref, w1_ref, w2_ref,
             b1_ref, b2_ref, fs_ref, t_ref):
    x = feat_ref[...]
    fs = lax.dot_general(x, wfc_ref[...], (((1,), (1,)), ((), ())),
                         preferred_element_type=jnp.float32)
    fs_ref[0] = fs[:, :128].astype(jnp.bfloat16)
    fs_ref[1] = fs[:, 128:].astype(jnp.bfloat16)
    h1 = lax.dot_general(ah_ref[...], w1_ref[...], (((1,), (1,)), ((), ())),
                         preferred_element_type=jnp.float32) + b1_ref[...]
    h1 = jnp.where(h1 > 0, h1, jnp.exp(h1) - 1.0)
    z = lax.dot_general(h1, w2_ref[...], (((1,), (0,)), ((), ())),
                        preferred_element_type=jnp.float32) + b2_ref[...]
    sig = 1.0 / (1.0 + jnp.exp(-z))
    cm8 = (lax.broadcasted_iota(jnp.int32, (1, 16), 1) == 8).astype(jnp.float32)
    ftp8 = sig * cm8
    t_ref[0] = lax.dot_general(fs, q0_ref[...], (((1,), (0,)), ((), ())),
                               preferred_element_type=jnp.float32) + ftp8
    t_ref[1] = lax.dot_general(fs, q1_ref[...], (((1,), (0,)), ((), ())),
                               preferred_element_type=jnp.float32) + ftp8


_tc_call = pl.pallas_call(
    _tc_body,
    grid=(_G,),
    in_specs=[
        pl.BlockSpec((_R, IN), lambda i: (i, 0)),
        pl.BlockSpec((_R, 128), lambda i: (i, 0)),
        pl.BlockSpec((H * F, IN), lambda i: (0, 0)),
        pl.BlockSpec((H * F, 16), lambda i: (0, 0)),
        pl.BlockSpec((H * F, 16), lambda i: (0, 0)),
        pl.BlockSpec((F, 128), lambda i: (0, 0)),
        pl.BlockSpec((F, 16), lambda i: (0, 0)),
        pl.BlockSpec((1, F), lambda i: (0, 0)),
        pl.BlockSpec((1, 16), lambda i: (0, 0)),
    ],
    out_specs=[
        pl.BlockSpec((2, _R, 128), lambda i: (0, i, 0)),
        pl.BlockSpec((2, _R, 16), lambda i: (0, i, 0)),
    ],
    out_shape=[
        jax.ShapeDtypeStruct((2, N, 128), jnp.bfloat16),
        jax.ShapeDtypeStruct((2, N, 16), jnp.float32),
    ],
)


def _sc_body(fstack, tstack, src_hbm, dst_hbm, bias_hbm,
             rst_hbm, alpha_hbm,
             idx_s, idx_d, idx_da, scd, ts, td, fsrc, stage, wblk,
             avblk, biasv, u_acc,
             sem_g0, sem_g1, sem_f0, sem_f1, sem_sc0, sem_sc1,
             sem_al0, sem_al1, sem_ix0, sem_ix1):
    c = lax.axis_index("c")
    s = lax.axis_index("s")
    cN = c * N
    c_f = c.astype(jnp.float32)
    sgn = 1.0 - 2.0 * c_f
    z16 = jnp.zeros((16,), jnp.float32)
    iota16 = lax.broadcasted_iota(jnp.int32, (16,), 0)
    semg = (sem_g0, sem_g1)
    semf = (sem_f0, sem_f1)
    semsc = (sem_sc0, sem_sc1)
    semal = (sem_al0, sem_al1)
    semix = (sem_ix0, sem_ix1)

    def _zrow(i, _):
        for p in range(2):
            for ch in range(8):
                stage[p, i, pl.ds(ch * 16, 16)] = z16
            stage[p, i, pl.ds(ROWW - 16, 16)] = z16
        return 0
    lax.fori_loop(0, KB, _zrow, 0)

    nch = jnp.where(s < NCH - (NCH // NSUB) * NSUB,
                    NCH // NSUB + 1, NCH // NSUB)

    def _zchunk(i, _):
        pltpu.sync_copy(stage.at[0, pl.ds(0, PCH)],
                        u_acc.at[pl.ds((s + i * NSUB) * PCH, PCH)])
        return 0
    lax.fori_loop(0, nch, _zchunk, 0)
    pltpu.sync_copy(bias_hbm.at[pl.ds(c * 128, 128)], biasv)
    plsc.subcore_barrier()

    def blk_of(j):
        return s + j * NSUB

    def idx_prefetch(j, q):
        off = blk_of(j) * KB
        pltpu.async_copy(src_hbm.at[pl.ds(off, KB)], idx_s.at[q], semix[q])
        pltpu.async_copy(dst_hbm.at[pl.ds(off, KB)], idx_d.at[q], semix[q])

    def gather_descs(q):
        return (
            pltpu.make_async_copy(tstack.at[idx_s.at[q]], ts.at[q], semg[q]),
            pltpu.make_async_copy(tstack.at[idx_da.at[q]], td.at[q], semg[q]),
            pltpu.make_async_copy(fstack.at[idx_s.at[q]], fsrc.at[q], semf[q]),
        )

    def fetch(j, q):
        @pl.when((j >= 2) & (blk_of(j - 2) < NBLK))
        def _():
            pltpu.make_async_copy(stage.at[q], u_acc.at[scd.at[q]],
                                  semsc[q]).wait()

            @pl.when(c == 0)
            def _():
                pltpu.make_async_copy(
                    avblk.at[q],
                    alpha_hbm.at[pl.ds(blk_of(j - 2) * KB, KB)],
                    semal[q]).wait()

        @pl.when(blk_of(j) < NBLK)
        def _():
            off = blk_of(j) * KB
            pltpu.make_async_copy(src_hbm.at[pl.ds(off, KB)], idx_s.at[q],
                                  semix[q]).wait()
            pltpu.make_async_copy(dst_hbm.at[pl.ds(off, KB)], idx_d.at[q],
                                  semix[q]).wait()
            for g in range(KB // 16):
                sl = pl.ds(g * 16, 16)
                idx_s[q, sl] = idx_s[q, sl] + cN
                idx_da[q, sl] = idx_d[q, sl] + cN
            for d in gather_descs(q):
                d.start()

        @pl.when(blk_of(j + 1) < NBLK)
        def _():
            idx_prefetch(j + 1, q ^ 1)

    def halfiter(i, p):
        b = blk_of(i)

        @pl.when(b < NBLK)
        def _():
            gd = gather_descs(p)
            gd[0].wait()
            gd[1].wait()
            for g in range(KB // 16):
                lanes = g * 16 + iota16
                fps = plsc.load_gather(ts.at[p], [lanes, jnp.full((16,), 8, jnp.int32)])
                fpd = plsc.load_gather(td.at[p], [lanes, jnp.full((16,), 8, jnp.int32)])
                sig = 1.0 / (1.0 + jnp.exp(-(fps + fpd)))
                av = c_f + sgn * sig
                avblk[p, pl.ds(g * 16, 16)] = sig
                for h in range(H2):
                    el = plsc.load_gather(ts.at[p], [lanes, jnp.full((16,), h, jnp.int32)])
                    er = plsc.load_gather(td.at[p], [lanes, jnp.full((16,), 4 + h, jnp.int32)])
                    x = el + er
                    xlr = jnp.where(x >= 0, x, 0.2 * x)
                    ex = jnp.exp(sgn * xlr)
                    plsc.store_scatter(stage.at[p], [lanes, jnp.full((16,), 128 + h, jnp.int32)], ex)
                    plsc.store_scatter(wblk, [lanes, jnp.full((16,), h, jnp.int32)], ex * av)
                sl = pl.ds(g * 16, 16)
                scd[p, sl] = idx_d[p, sl]

        fetch(i + 1, p ^ 1)

        @pl.when(b < NBLK)
        def _():
            pltpu.make_async_copy(fstack.at[idx_s.at[p]], fsrc.at[p],
                                  semf[p]).wait()
            def _row(r2, _):
                r = r2 * 2
                wva = wblk[r, pl.ds(0, 16)]
                wvb = wblk[r + 1, pl.ds(0, 16)]
                xa = [fsrc[p, r, pl.ds(k * 32, 32)] for k in range(4)]
                xb = [fsrc[p, r + 1, pl.ds(k * 32, 32)] for k in range(4)]
                for h in range(H2):
                    a0, a1 = plsc.unpack(xa[h], format=plsc.PackFormat.INTERLEAVED)
                    b0, b1 = plsc.unpack(xb[h], format=plsc.PackFormat.INTERLEAVED)
                    wa = wva[h]
                    wb = wvb[h]
                    stage[p, r, pl.ds(h * 32, 16)] = a0 * wa
                    stage[p, r, pl.ds(h * 32 + 16, 16)] = a1 * wa
                    stage[p, r + 1, pl.ds(h * 32, 16)] = b0 * wb
                    stage[p, r + 1, pl.ds(h * 32 + 16, 16)] = b1 * wb
                return 0
            lax.fori_loop(0, KB // 2, _row, 0)
            pltpu.async_copy(stage.at[p], u_acc.at[scd.at[p]], semsc[p],
                             add=True)

            @pl.when(c == 0)
            def _():
                pltpu.async_copy(avblk.at[p],
                                 alpha_hbm.at[pl.ds(b * KB, KB)], semal[p])

    idx_prefetch(0, 0)
    fetch(0, 0)

    def _pair(t, _):
        halfiter(2 * t, 0)
        halfiter(2 * t + 1, 1)
        return 0
    lax.fori_loop(0, NITER // 2, _pair, 0)
    plsc.subcore_barrier()

    def _fchunk(i, _):
        base = (s + i * NSUB) * PCH
        pltpu.sync_copy(u_acc.at[pl.ds(base, PCH)], stage.at[0, pl.ds(0, PCH)])

        def _node(r, _):
            sv = stage[0, r, pl.ds(ROWW - 16, 16)]
            rv = jnp.where(sv > 0.0, 1.0 / sv, 0.0)
            uv = [stage[0, r, pl.ds(k * 16, 16)] for k in range(8)]
            for h in range(H2):
                rin = rv[8 + h]
                for v in range(2):
                    cl = pl.ds(h * 32 + v * 16, 16)
                    stage[1, r, cl] = uv[h * 2 + v] * rin + biasv[cl]
            return 0
        lax.fori_loop(0, PCH, _node, 0)
        pltpu.sync_copy(stage.at[1, pl.ds(0, PCH), pl.ds(0, 128)],
                        rst_hbm.at[pl.ds(base, PCH), pl.ds(c * 128, 128)])
        return 0
    lax.fori_loop(0, nch, _fchunk, 0)


def _make_sc():
    mesh = plsc.VectorSubcoreMesh(core_axis_name="c", subcore_axis_name="s",
                                  num_cores=2, num_subcores=NSUB)
    return pl.kernel(
        _sc_body,
        out_type=(jax.ShapeDtypeStruct((N, 2 * 128), jnp.float32),
                  jax.ShapeDtypeStruct((E,), jnp.float32)),
        mesh=mesh,
        compiler_params=pltpu.CompilerParams(use_tc_tiling_on_sc=False,
                                             needs_layout_passes=False),
        scratch_types=dict(
            idx_s=pltpu.VMEM((2, KB), jnp.int32),
            idx_d=pltpu.VMEM((2, KB), jnp.int32),
            idx_da=pltpu.VMEM((2, KB), jnp.int32),
            scd=pltpu.VMEM((2, KB), jnp.int32),
            ts=pltpu.VMEM((2, KB, 16), jnp.float32),
            td=pltpu.VMEM((2, KB, 16), jnp.float32),
            fsrc=pltpu.VMEM((2, KB, 128), jnp.bfloat16),
            stage=pltpu.VMEM((2, KB, ROWW), jnp.float32),
            wblk=pltpu.VMEM((KB, 16), jnp.float32),
            avblk=pltpu.VMEM((2, KB), jnp.float32),
            biasv=pltpu.VMEM((128,), jnp.float32),
            u_acc=pltpu.MemorySpace.VMEM_SHARED((N, ROWW), jnp.float32),
            sem_g0=pltpu.SemaphoreType.DMA,
            sem_g1=pltpu.SemaphoreType.DMA,
            sem_f0=pltpu.SemaphoreType.DMA,
            sem_f1=pltpu.SemaphoreType.DMA,
            sem_sc0=pltpu.SemaphoreType.DMA,
            sem_sc1=pltpu.SemaphoreType.DMA,
            sem_al0=pltpu.SemaphoreType.DMA,
            sem_al1=pltpu.SemaphoreType.DMA,
            sem_ix0=pltpu.SemaphoreType.DMA,
            sem_ix1=pltpu.SemaphoreType.DMA,
        ),
    )


def kernel(feat, alpha_hidden, edge_index, W_fc, attn_l, attn_r, attn_ln,
           attn_rn, bias, W1, b1, W2, b2):
    eye4 = jnp.eye(4, dtype=jnp.float32)
    def bd(a):
        return (a[:, :, None] * eye4[:, None, :]).reshape(128, 4)
    q0 = jnp.zeros((H * F, 16), jnp.float32)
    q0 = q0.at[0:128, 0:4].set(bd(attn_l[0])).at[0:128, 4:8].set(bd(attn_r[0]))
    q1 = jnp.zeros((H * F, 16), jnp.float32)
    q1 = q1.at[128:256, 0:4].set(bd(attn_ln[0])).at[128:256, 4:8].set(bd(attn_rn[0]))

    pidx = np.zeros((H * F,), np.int32)
    for h in range(H):
        for j in range(16):
            pidx[h * 32 + 2 * j] = h * 32 + j
            pidx[h * 32 + 2 * j + 1] = h * 32 + 16 + j
    pidx = jnp.asarray(pidx)
    W_fc = W_fc[pidx]
    q0 = q0[pidx]
    q1 = q1[pidx]

    w2p = jnp.zeros((F, 16), jnp.float32).at[:, 8].set(W2[0])
    b2r = jnp.broadcast_to(b2.reshape(1, 1), (1, 16))
    fs2, t2 = _tc_call(feat, alpha_hidden, W_fc, q0, q1, W1,
                       w2p, b1.reshape(1, F), b2r)
    fstack = fs2.reshape(2 * N, 128)
    tstack = t2.reshape(2 * N, 16)

    src = edge_index[0]
    dst = edge_index[1]
    rst_flat, alpha = _make_sc()(fstack, tstack, src, dst, bias)
    return (rst_flat.reshape(N, H, F), alpha.reshape(E, 1, 1))


# speedup vs baseline: 1.0793x; 1.0465x over previous
import functools

import jax
import jax.numpy as jnp
import numpy as np
from jax import lax
from jax.experimental import pallas as pl
from jax.experimental.pallas import tpu as pltpu
from jax.experimental.pallas import tpu_sc as plsc

N = 10000
E = 320000
IN = 128
H = 8
F = 32
H2 = H // 2

KB = 80
NBLK = E // KB
NSUB = 16
ROWW = 136
PCH = 50
NCH = N // PCH
NITER = NBLK // NSUB + 2

_R = 400
_G = N // _R


def _tc_body(feat_ref, ah_ref, wfc_ref, q0_ref, q1_ref, w1_ref, w2_ref,
             b1_ref, b2_ref, fs_ref, t_ref):
    x = feat_ref[...]
    fs = lax.dot_general(x, wfc_ref[...], (((1,), (1,)), ((), ())),
                         preferred_element_type=jnp.float32)
    fs_ref[0] = fs[:, :128].astype(jnp.bfloat16)
    fs_ref[1] = fs[:, 128:].astype(jnp.bfloat16)
    h1 = lax.dot_general(ah_ref[...], w1_ref[...], (((1,), (1,)), ((), ())),
                         preferred_element_type=jnp.float32) + b1_ref[...]
    h1 = jnp.where(h1 > 0, h1, jnp.exp(h1) - 1.0)
    z = lax.dot_general(h1, w2_ref[...], (((1,), (0,)), ((), ())),
                        preferred_element_type=jnp.float32) + b2_ref[...]
    sig = 1.0 / (1.0 + jnp.exp(-z))
    cm8 = (lax.broadcasted_iota(jnp.int32, (1, 16), 1) == 8).astype(jnp.float32)
    ftp8 = sig * cm8
    t_ref[0] = lax.dot_general(fs, q0_ref[...], (((1,), (0,)), ((), ())),
                               preferred_element_type=jnp.float32) + ftp8
    t_ref[1] = lax.dot_general(fs, q1_ref[...], (((1,), (0,)), ((), ())),
                               preferred_element_type=jnp.float32) + ftp8


_tc_call = pl.pallas_call(
    _tc_body,
    grid=(_G,),
    in_specs=[
        pl.BlockSpec((_R, IN), lambda i: (i, 0)),
        pl.BlockSpec((_R, 128), lambda i: (i, 0)),
        pl.BlockSpec((H * F, IN), lambda i: (0, 0)),
        pl.BlockSpec((H * F, 16), lambda i: (0, 0)),
        pl.BlockSpec((H * F, 16), lambda i: (0, 0)),
        pl.BlockSpec((F, 128), lambda i: (0, 0)),
        pl.BlockSpec((F, 16), lambda i: (0, 0)),
        pl.BlockSpec((1, F), lambda i: (0, 0)),
        pl.BlockSpec((1, 16), lambda i: (0, 0)),
    ],
    out_specs=[
        pl.BlockSpec((2, _R, 128), lambda i: (0, i, 0)),
        pl.BlockSpec((2, _R, 16), lambda i: (0, i, 0)),
    ],
    out_shape=[
        jax.ShapeDtypeStruct((2, N, 128), jnp.bfloat16),
        jax.ShapeDtypeStruct((2, N, 16), jnp.float32),
    ],
)


def _sc_body(fstack, tstack, src_hbm, dst_hbm, bias_hbm,
             rst_hbm, alpha_hbm,
             idx_s, idx_d, idx_da, scd, ts, td, fsrc, stage, wblk,
             avblk, biasv, u_acc,
             sem_g0, sem_g1, sem_f0, sem_f1, sem_sc0, sem_sc1,
             sem_al0, sem_al1, sem_ix0, sem_ix1):
    c = lax.axis_index("c")
    s = lax.axis_index("s")
    cN = c * N
    c_f = c.astype(jnp.float32)
    sgn = 1.0 - 2.0 * c_f
    z16 = jnp.zeros((16,), jnp.float32)
    iota16 = lax.broadcasted_iota(jnp.int32, (16,), 0)
    semg = (sem_g0, sem_g1)
    semf = (sem_f0, sem_f1)
    semsc = (sem_sc0, sem_sc1)
    semal = (sem_al0, sem_al1)
    semix = (sem_ix0, sem_ix1)

    def _zrow(i, _):
        for p in range(2):
            for ch in range(8):
                stage[p, i, pl.ds(ch * 16, 16)] = z16
            stage[p, i, pl.ds(ROWW - 16, 16)] = z16
        return 0
    lax.fori_loop(0, KB, _zrow, 0)

    nch = jnp.where(s < NCH - (NCH // NSUB) * NSUB,
                    NCH // NSUB + 1, NCH // NSUB)

    def _zchunk(i, _):
        pltpu.sync_copy(stage.at[0, pl.ds(0, PCH)],
                        u_acc.at[pl.ds((s + i * NSUB) * PCH, PCH)])
        return 0
    lax.fori_loop(0, nch, _zchunk, 0)
    pltpu.sync_copy(bias_hbm.at[pl.ds(c * 128, 128)], biasv)
    plsc.subcore_barrier()

    def blk_of(j):
        return s + j * NSUB

    def idx_prefetch(j, q):
        off = blk_of(j) * KB
        pltpu.async_copy(src_hbm.at[pl.ds(off, KB)], idx_s.at[q], semix[q])
        pltpu.async_copy(dst_hbm.at[pl.ds(off, KB)], idx_d.at[q], semix[q])

    def gather_descs(q):
        return (
            pltpu.make_async_copy(tstack.at[idx_s.at[q]], ts.at[q], semg[q]),
            pltpu.make_async_copy(tstack.at[idx_da.at[q]], td.at[q], semg[q]),
            pltpu.make_async_copy(fstack.at[idx_s.at[q]], fsrc.at[q], semf[q]),
        )

    def fetch(j, q):
        @pl.when((j >= 2) & (blk_of(j - 2) < NBLK))
        def _():
            pltpu.make_async_copy(stage.at[q], u_acc.at[scd.at[q]],
                                  semsc[q]).wait()

            @pl.when(c == 0)
            def _():
                pltpu.make_async_copy(
                    avblk.at[q],
                    alpha_hbm.at[pl.ds(blk_of(j - 2) * KB, KB)],
                    semal[q]).wait()

        @pl.when(blk_of(j) < NBLK)
        def _():
            off = blk_of(j) * KB
            pltpu.make_async_copy(src_hbm.at[pl.ds(off, KB)], idx_s.at[q],
                                  semix[q]).wait()
            pltpu.make_async_copy(dst_hbm.at[pl.ds(off, KB)], idx_d.at[q],
                                  semix[q]).wait()
            for g in range(KB // 16):
                sl = pl.ds(g * 16, 16)
                idx_s[q, sl] = idx_s[q, sl] + cN
                idx_da[q, sl] = idx_d[q, sl] + cN
            for d in gather_descs(q):
                d.start()

        @pl.when(blk_of(j + 1) < NBLK)
        def _():
            idx_prefetch(j + 1, q ^ 1)

    def halfiter(i, p):
        b = blk_of(i)

        @pl.when(b < NBLK)
        def _():
            gd = gather_descs(p)
            gd[0].wait()
            gd[1].wait()
            for g in range(KB // 16):
                lanes = g * 16 + iota16
                fps = plsc.load_gather(ts.at[p], [lanes, jnp.full((16,), 8, jnp.int32)])
                fpd = plsc.load_gather(td.at[p], [lanes, jnp.full((16,), 8, jnp.int32)])
                sig = 1.0 / (1.0 + jnp.exp(-(fps + fpd)))
                av = c_f + sgn * sig
                avblk[p, pl.ds(g * 16, 16)] = sig
                for h in range(H2):
                    el = plsc.load_gather(ts.at[p], [lanes, jnp.full((16,), h, jnp.int32)])
                    er = plsc.load_gather(td.at[p], [lanes, jnp.full((16,), 4 + h, jnp.int32)])
                    x = el + er
                    xlr = jnp.where(x >= 0, x, 0.2 * x)
                    ex = jnp.exp(sgn * xlr)
                    plsc.store_scatter(stage.at[p], [lanes, jnp.full((16,), 128 + h, jnp.int32)], ex)
                    plsc.store_scatter(wblk, [lanes, jnp.full((16,), h, jnp.int32)], ex * av)
                sl = pl.ds(g * 16, 16)
                scd[p, sl] = idx_d[p, sl]

        fetch(i + 1, p ^ 1)

        @pl.when(b < NBLK)
        def _():
            pltpu.make_async_copy(fstack.at[idx_s.at[p]], fsrc.at[p],
                                  semf[p]).wait()
            def _row(r2, _):
                r = r2 * 2
                wva = wblk[r, pl.ds(0, 16)]
                wvb = wblk[r + 1, pl.ds(0, 16)]
                xa = [fsrc[p, r, pl.ds(k * 32, 32)] for k in range(4)]
                xb = [fsrc[p, r + 1, pl.ds(k * 32, 32)] for k in range(4)]
                for h in range(H2):
                    a0, a1 = plsc.unpack(xa[h], format=plsc.PackFormat.INTERLEAVED)
                    b0, b1 = plsc.unpack(xb[h], format=plsc.PackFormat.INTERLEAVED)
                    wa = wva[h]
                    wb = wvb[h]
                    stage[p, r, pl.ds(h * 32, 16)] = a0 * wa
                    stage[p, r, pl.ds(h * 32 + 16, 16)] = a1 * wa
                    stage[p, r + 1, pl.ds(h * 32, 16)] = b0 * wb
                    stage[p, r + 1, pl.ds(h * 32 + 16, 16)] = b1 * wb
                return 0
            lax.fori_loop(0, KB // 2, _row, 0)
            pltpu.async_copy(stage.at[p], u_acc.at[scd.at[p]], semsc[p],
                             add=True)

            @pl.when(c == 0)
            def _():
                pltpu.async_copy(avblk.at[p],
                                 alpha_hbm.at[pl.ds(b * KB, KB)], semal[p])

    idx_prefetch(0, 0)
    fetch(0, 0)

    def _pair(t, _):
        halfiter(2 * t, 0)
        halfiter(2 * t + 1, 1)
        return 0
    lax.fori_loop(0, NITER // 2, _pair, 0)
    plsc.subcore_barrier()

    def _fchunk(i, _):
        base = (s + i * NSUB) * PCH
        pltpu.sync_copy(u_acc.at[pl.ds(base, PCH)], stage.at[0, pl.ds(0, PCH)])

        def _node(r, _):
            sv = stage[0, r, pl.ds(ROWW - 16, 16)]
            rv = jnp.where(sv > 0.0, 1.0 / sv, 0.0)
            uv = [stage[0, r, pl.ds(k * 16, 16)] for k in range(8)]
            for h in range(H2):
                rin = rv[8 + h]
                for v in range(2):
                    cl = pl.ds(h * 32 + v * 16, 16)
                    stage[1, r, cl] = uv[h * 2 + v] * rin + biasv[cl]
            return 0
        lax.fori_loop(0, PCH, _node, 0)
        pltpu.sync_copy(stage.at[1, pl.ds(0, PCH), pl.ds(0, 128)],
                        rst_hbm.at[pl.ds(base, PCH), pl.ds(c * 128, 128)])
        return 0
    lax.fori_loop(0, nch, _fchunk, 0)


def _make_sc():
    mesh = plsc.VectorSubcoreMesh(core_axis_name="c", subcore_axis_name="s",
                                  num_cores=2, num_subcores=NSUB)
    return pl.kernel(
        _sc_body,
        out_type=(jax.ShapeDtypeStruct((N, 2 * 128), jnp.float32),
                  jax.ShapeDtypeStruct((E,), jnp.float32)),
        mesh=mesh,
        compiler_params=pltpu.CompilerParams(use_tc_tiling_on_sc=False,
                                             needs_layout_passes=False),
        scratch_types=dict(
            idx_s=pltpu.VMEM((2, KB), jnp.int32),
            idx_d=pltpu.VMEM((2, KB), jnp.int32),
            idx_da=pltpu.VMEM((2, KB), jnp.int32),
            scd=pltpu.VMEM((2, KB), jnp.int32),
            ts=pltpu.VMEM((2, KB, 16), jnp.float32),
            td=pltpu.VMEM((2, KB, 16), jnp.float32),
            fsrc=pltpu.VMEM((2, KB, 128), jnp.bfloat16),
            stage=pltpu.VMEM((2, KB, ROWW), jnp.float32),
            wblk=pltpu.VMEM((KB, 16), jnp.float32),
            avblk=pltpu.VMEM((2, KB), jnp.float32),
            biasv=pltpu.VMEM((128,), jnp.float32),
            u_acc=pltpu.MemorySpace.VMEM_SHARED((N, ROWW), jnp.float32),
            sem_g0=pltpu.SemaphoreType.DMA,
            sem_g1=pltpu.SemaphoreType.DMA,
            sem_f0=pltpu.SemaphoreType.DMA,
            sem_f1=pltpu.SemaphoreType.DMA,
            sem_sc0=pltpu.SemaphoreType.DMA,
            sem_sc1=pltpu.SemaphoreType.DMA,
            sem_al0=pltpu.SemaphoreType.DMA,
            sem_al1=pltpu.SemaphoreType.DMA,
            sem_ix0=pltpu.SemaphoreType.DMA,
            sem_ix1=pltpu.SemaphoreType.DMA,
        ),
    )


def kernel(feat, alpha_hidden, edge_index, W_fc, attn_l, attn_r, attn_ln,
           attn_rn, bias, W1, b1, W2, b2):
    eye4 = jnp.eye(4, dtype=jnp.float32)
    def bd(a):
        return (a[:, :, None] * eye4[:, None, :]).reshape(128, 4)
    q0 = jnp.zeros((H * F, 16), jnp.float32)
    q0 = q0.at[0:128, 0:4].set(bd(attn_l[0])).at[0:128, 4:8].set(bd(attn_r[0]))
    q1 = jnp.zeros((H * F, 16), jnp.float32)
    q1 = q1.at[128:256, 0:4].set(bd(attn_ln[0])).at[128:256, 4:8].set(bd(attn_rn[0]))

    pidx = np.zeros((H * F,), np.int32)
    for h in range(H):
        for j in range(16):
            pidx[h * 32 + 2 * j] = h * 32 + j
            pidx[h * 32 + 2 * j + 1] = h * 32 + 16 + j
    pidx = jnp.asarray(pidx)
    W_fc = W_fc[pidx]
    q0 = q0[pidx]
    q1 = q1[pidx]

    w2p = jnp.zeros((F, 16), jnp.float32).at[:, 8].set(W2[0])
    b2r = jnp.broadcast_to(b2.reshape(1, 1), (1, 16))
    fs2, t2 = _tc_call(feat, alpha_hidden, W_fc, q0, q1, W1,
                       w2p, b1.reshape(1, F), b2r)
    fstack = fs2.reshape(2 * N, 128)
    tstack = t2.reshape(2 * N, 16)

    src = edge_index[0]
    dst = edge_index[1]
    rst_flat, alpha = _make_sc()(fstack, tstack, src, dst, bias)
    return (rst_flat.reshape(N, H, F), alpha.reshape(E, 1, 1))
